# Initial kernel scaffold; baseline (speedup 1.0000x reference)
import jax
import jax.numpy as jnp
from jax.experimental import pallas as pl


def kernel(pred, target):
    raise NotImplementedError("write your pallas kernel here")


# speedup vs baseline: 1.6563x; 1.6563x over previous
import functools

import jax
import jax.numpy as jnp
from jax.experimental import pallas as pl
from jax.experimental.pallas import tpu as pltpu

_GHM = 0.75
_NBINS = 10
_ROWS, _COLS = 16384, 1024
_BLK = 512
_NSTEPS = _ROWS // _BLK
_EDGES = [float(i) / _NBINS for i in range(1, _NBINS)]


def _body(pred_ref, tgt_ref, out_ref, acc_ref):
    step = pl.program_id(0)

    @pl.when(step == 0)
    def _init():
        for i in range(20):
            acc_ref[i] = 0.0

    p = pred_ref[...]
    t = tgt_ref[...].astype(jnp.float32)
    u = jnp.exp(-jnp.abs(p))
    w = 1.0 + u
    r = 1.0 / w
    sig = jnp.where(p >= 0.0, r, u * r)
    g = jnp.abs(sig - t)
    loss_e = jnp.maximum(p, 0.0) - p * t + jnp.log(w)

    acc_ref[19] += jnp.sum(loss_e)
    for i, e in enumerate(_EDGES):
        m = g < e
        acc_ref[i] += jnp.sum(jnp.where(m, 1.0, 0.0))
        acc_ref[9 + i] += jnp.sum(jnp.where(m, loss_e, 0.0))

    @pl.when(step == _NSTEPS - 1)
    def _fini():
        total = float(_ROWS * _COLS)
        lt = [0.0] + [acc_ref[i] for i in range(9)] + [total]
        wlt = [0.0] + [acc_ref[9 + i] for i in range(9)] + [acc_ref[19]]
        loss = 0.0
        for i in range(_NBINS):
            c = lt[i + 1] - lt[i]
            s = wlt[i + 1] - wlt[i]
            loss += s / jnp.maximum(c, 1.0)
        out_ref[0] = loss * (_GHM / 10.0)


def kernel(pred, target):
    out = pl.pallas_call(
        _body,
        grid=(_NSTEPS,),
        in_specs=[
            pl.BlockSpec((_BLK, _COLS), lambda i: (i, 0)),
            pl.BlockSpec((_BLK, _COLS), lambda i: (i, 0)),
        ],
        out_specs=pl.BlockSpec(memory_space=pltpu.SMEM),
        out_shape=jax.ShapeDtypeStruct((1,), jnp.float32),
        scratch_shapes=[pltpu.SMEM((20,), jnp.float32)],
    )(pred, target)
    return out.reshape(())


# speedup vs baseline: 2.0621x; 1.2450x over previous
import functools
import math

import jax
import jax.numpy as jnp
from jax.experimental import pallas as pl
from jax.experimental.pallas import tpu as pltpu

_GHM = 0.75
_NBINS = 10
_ROWS, _COLS = 16384, 1024
_BLK = 512
_NSTEPS = _ROWS // _BLK
_LOGIT_EDGES = [math.log(i / (_NBINS - i)) for i in range(1, _NBINS)]


def _body(pred_ref, tgt_ref, out_ref, acc_ref):
    step = pl.program_id(0)

    @pl.when(step == 0)
    def _init():
        for i in range(20):
            acc_ref[i] = 0.0

    p = pred_ref[...]
    t = tgt_ref[...].astype(jnp.float32)
    sp = p * (1.0 - 2.0 * t)
    u = jnp.exp(-jnp.abs(sp))
    loss_e = jnp.maximum(sp, 0.0) + jnp.log(1.0 + u)

    acc_ref[19] += jnp.sum(loss_e)
    for i, le in enumerate(_LOGIT_EDGES):
        mf = (sp < le).astype(jnp.float32)
        acc_ref[i] += jnp.sum(mf)
        acc_ref[9 + i] += jnp.sum(mf * loss_e)

    @pl.when(step == _NSTEPS - 1)
    def _fini():
        total = float(_ROWS * _COLS)
        lt = [0.0] + [acc_ref[i] for i in range(9)] + [total]
        wlt = [0.0] + [acc_ref[9 + i] for i in range(9)] + [acc_ref[19]]
        loss = 0.0
        for i in range(_NBINS):
            c = lt[i + 1] - lt[i]
            s = wlt[i + 1] - wlt[i]
            loss += s / jnp.maximum(c, 1.0)
        out_ref[0] = loss * (_GHM / 10.0)


def kernel(pred, target):
    out = pl.pallas_call(
        _body,
        grid=(_NSTEPS,),
        in_specs=[
            pl.BlockSpec((_BLK, _COLS), lambda i: (i, 0)),
            pl.BlockSpec((_BLK, _COLS), lambda i: (i, 0)),
        ],
        out_specs=pl.BlockSpec(memory_space=pltpu.SMEM),
        out_shape=jax.ShapeDtypeStruct((1,), jnp.float32),
        scratch_shapes=[pltpu.SMEM((20,), jnp.float32)],
    )(pred, target)
    return out.reshape(())
